# Initial kernel scaffold; baseline (speedup 1.0000x reference)
import jax
import jax.numpy as jnp
from jax.experimental import pallas as pl


def kernel(x, table):
    raise NotImplementedError("write your pallas kernel here")


# speedup vs baseline: 114.4747x; 114.4747x over previous
import functools

import jax
import jax.numpy as jnp
from jax import lax
from jax.experimental import pallas as pl
from jax.experimental.pallas import tpu as pltpu
from jax.experimental.pallas import tpu_sc as plsc


def _exp2_table_body(t_ref, o_ref):
    v = jnp.exp2(t_ref[...])
    rows = lax.broadcasted_iota(jnp.int32, v.shape, 0)
    cols = lax.broadcasted_iota(jnp.int32, v.shape, 1)
    frozen = (rows == 0) & (cols == 0)
    o_ref[...] = jnp.where(frozen, 1.0, v)


def _exp2_table(table_2d):
    return pl.pallas_call(
        _exp2_table_body,
        out_shape=jax.ShapeDtypeStruct(table_2d.shape, jnp.float32),
    )(table_2d)


_INFO = plsc.get_sparse_core_info()
_NC = _INFO.num_cores
_NS = _INFO.num_subcores
_NW = _NC * _NS


@functools.cache
def _make_gather(n, chunk):
    per_w = n // _NW
    n_chunks = per_w // chunk
    mesh = plsc.VectorSubcoreMesh(core_axis_name="c", subcore_axis_name="s")

    @functools.partial(
        pl.kernel,
        mesh=mesh,
        out_type=jax.ShapeDtypeStruct((n,), jnp.float32),
        scratch_types=[
            pltpu.VMEM((chunk,), jnp.int32),
            pltpu.VMEM((chunk,), jnp.float32),
            pltpu.SemaphoreType.DMA,
        ],
    )
    def gather_kernel(table_hbm, idx_hbm, out_hbm, idx_v, rows_v, sem):
        wid = lax.axis_index("s") * _NC + lax.axis_index("c")
        base = wid * per_w

        def body(i, carry):
            off = base + i * chunk
            pltpu.sync_copy(idx_hbm.at[pl.ds(off, chunk)], idx_v)
            pltpu.async_copy(table_hbm.at[idx_v], rows_v, sem).wait()
            pltpu.sync_copy(rows_v, out_hbm.at[pl.ds(off, chunk)])
            return carry

        lax.fori_loop(0, n_chunks, body, 0)

    return gather_kernel


def kernel(x, table):
    b, l = x.shape
    n = b * l
    v = table.shape[0]
    v_pad = -(-v // 1024) * 1024
    t = jnp.pad(table.reshape(-1), (0, v_pad - v)).reshape(-1, 1024)
    table2 = _exp2_table(t).reshape(-1)
    out = _make_gather(n, 12800)(table2, x.reshape(-1))
    return out.reshape(b, l, 1)


# speedup vs baseline: 115.9854x; 1.0132x over previous
import functools

import jax
import jax.numpy as jnp
from jax import lax
from jax.experimental import pallas as pl
from jax.experimental.pallas import tpu as pltpu
from jax.experimental.pallas import tpu_sc as plsc

_INFO = plsc.get_sparse_core_info()
_NC = _INFO.num_cores
_NS = _INFO.num_subcores
_NW = _NC * _NS
_LN2 = 0.6931471805599453


@functools.cache
def _make_gather(n, chunk):
    per_w = n // _NW
    n_chunks = per_w // chunk
    assert n_chunks >= 2 and per_w % chunk == 0
    mesh = plsc.VectorSubcoreMesh(core_axis_name="c", subcore_axis_name="s")

    @functools.partial(
        pl.kernel,
        mesh=mesh,
        out_type=jax.ShapeDtypeStruct((n,), jnp.float32),
        scratch_types=[
            pltpu.VMEM((chunk,), jnp.int32),
            pltpu.VMEM((chunk,), jnp.int32),
            pltpu.VMEM((chunk,), jnp.float32),
            pltpu.VMEM((chunk,), jnp.float32),
            pltpu.SemaphoreType.DMA,
            pltpu.SemaphoreType.DMA,
            pltpu.SemaphoreType.DMA,
            pltpu.SemaphoreType.DMA,
            pltpu.SemaphoreType.DMA,
            pltpu.SemaphoreType.DMA,
        ],
    )
    def gather_kernel(table_hbm, idx_hbm, out_hbm,
                      idx0, idx1, rows0, rows1,
                      si0, si1, sg0, sg1, so0, so1):
        wid = lax.axis_index("s") * _NC + lax.axis_index("c")
        base = wid * per_w
        idx_v = [idx0, idx1]
        rows_v = [rows0, rows1]
        si = [si0, si1]
        sg = [sg0, sg1]
        so = [so0, so1]
        fetches = [None, None]
        gathers = [None, None]
        stores = [None, None]

        def compute(b):
            iv, rv = idx_v[b], rows_v[b]

            def body(k, carry):
                s = pl.ds(k * 16, 16)
                t = rv[s] * _LN2
                r = jnp.where(iv[s] == 0, 1.0, jnp.exp(t))
                rv[s] = r
                return carry

            lax.fori_loop(0, chunk // 16, body, 0)

        for i in range(2):
            fetches[i] = pltpu.async_copy(
                idx_hbm.at[pl.ds(base + i * chunk, chunk)], idx_v[i], si[i])
        fetches[0].wait()
        gathers[0] = pltpu.async_copy(
            table_hbm.at[idx_v[0]], rows_v[0], sg[0])

        for i in range(n_chunks):
            b, nb = i % 2, (i + 1) % 2
            if i + 1 < n_chunks:
                if stores[nb] is not None:
                    stores[nb].wait()
                fetches[nb].wait()
                gathers[nb] = pltpu.async_copy(
                    table_hbm.at[idx_v[nb]], rows_v[nb], sg[nb])
            gathers[b].wait()
            compute(b)
            stores[b] = pltpu.async_copy(
                rows_v[b], out_hbm.at[pl.ds(base + i * chunk, chunk)], so[b])
            if i + 2 < n_chunks:
                fetches[b] = pltpu.async_copy(
                    idx_hbm.at[pl.ds(base + (i + 2) * chunk, chunk)],
                    idx_v[b], si[b])
        stores[0].wait()
        stores[1].wait()

    return gather_kernel


def kernel(x, table):
    b, l = x.shape
    n = b * l
    out = _make_gather(n, 12800)(table.reshape(-1), x.reshape(-1))
    return out.reshape(b, l, 1)


# speedup vs baseline: 144.9009x; 1.2493x over previous
import functools

import jax
import jax.numpy as jnp
from jax import lax
from jax.experimental import pallas as pl
from jax.experimental.pallas import tpu as pltpu
from jax.experimental.pallas import tpu_sc as plsc

_INFO = plsc.get_sparse_core_info()
_NC = _INFO.num_cores
_NS = _INFO.num_subcores
_NW = _NC * _NS
_LN2 = 0.6931471805599453


@functools.cache
def _make_gather(n, v, chunk):
    per_w = n // _NW
    n_chunks = per_w // chunk
    assert n_chunks >= 2 and per_w % chunk == 0
    n_full = v // chunk
    tail_off = n_full * chunk
    tail = v - tail_off
    assert tail % 16 == 0
    plan = [[] for _ in range(_NS)]
    for c in range(n_full):
        plan[c % _NS].append((c * chunk, chunk))
    if tail:
        plan[_NS - 1].append((tail_off, tail))
    mesh = plsc.VectorSubcoreMesh(core_axis_name="c", subcore_axis_name="s")

    @functools.partial(
        pl.kernel,
        mesh=mesh,
        out_type=jax.ShapeDtypeStruct((n,), jnp.float32),
        scratch_types=[
            pltpu.VMEM_SHARED((v,), jnp.float32),
            pltpu.VMEM((chunk,), jnp.int32),
            pltpu.VMEM((chunk,), jnp.int32),
            pltpu.VMEM((chunk,), jnp.float32),
            pltpu.VMEM((chunk,), jnp.float32),
            pltpu.SemaphoreType.DMA,
            pltpu.SemaphoreType.DMA,
            pltpu.SemaphoreType.DMA,
            pltpu.SemaphoreType.DMA,
            pltpu.SemaphoreType.DMA,
            pltpu.SemaphoreType.DMA,
        ],
    )
    def gather_kernel(table_hbm, idx_hbm, out_hbm,
                      shared,
                      idx0, idx1, rows0, rows1,
                      si0, si1, sg0, sg1, so0, so1):
        tid = lax.axis_index("s")
        cid = lax.axis_index("c")
        wid = tid * _NC + cid
        base = wid * per_w
        idx_v = [idx0, idx1]
        rows_v = [rows0, rows1]
        si = [si0, si1]
        sg = [sg0, sg1]
        so = [so0, so1]
        fetches = [None, None]
        gathers = [None, None]
        stores = [None, None]

        for i in range(2):
            fetches[i] = pltpu.async_copy(
                idx_hbm.at[pl.ds(base + i * chunk, chunk)], idx_v[i], si[i])

        def stage(off, size):
            pltpu.sync_copy(table_hbm.at[pl.ds(off, size)],
                            rows0.at[pl.ds(0, size)])

            def body(k, carry):
                s = pl.ds(k * 16, 16)
                g = lax.iota(jnp.int32, 16) + (off + k * 16)
                rows0[s] = jnp.where(g == 0, 1.0, jnp.exp(rows0[s] * _LN2))
                return carry

            lax.fori_loop(0, size // 16, body, 0)
            pltpu.sync_copy(rows0.at[pl.ds(0, size)],
                            shared.at[pl.ds(off, size)])

        for t in range(_NS):
            @pl.when(tid == t)
            def _(t=t):
                for off, size in plan[t]:
                    stage(off, size)

        plsc.subcore_barrier()

        fetches[0].wait()
        gathers[0] = pltpu.async_copy(shared.at[idx_v[0]], rows_v[0], sg[0])

        for i in range(n_chunks):
            b, nb = i % 2, (i + 1) % 2
            if i + 1 < n_chunks:
                if stores[nb] is not None:
                    stores[nb].wait()
                fetches[nb].wait()
                gathers[nb] = pltpu.async_copy(
                    shared.at[idx_v[nb]], rows_v[nb], sg[nb])
            gathers[b].wait()
            stores[b] = pltpu.async_copy(
                rows_v[b], out_hbm.at[pl.ds(base + i * chunk, chunk)], so[b])
            if i + 2 < n_chunks:
                fetches[b] = pltpu.async_copy(
                    idx_hbm.at[pl.ds(base + (i + 2) * chunk, chunk)],
                    idx_v[b], si[b])
        stores[0].wait()
        stores[1].wait()

    return gather_kernel


def kernel(x, table):
    b, l = x.shape
    n = b * l
    v = table.shape[0]
    out = _make_gather(n, v, 12800)(table.reshape(-1), x.reshape(-1))
    return out.reshape(b, l, 1)


# speedup vs baseline: 203.7545x; 1.4062x over previous
import functools

import jax
import jax.numpy as jnp
from jax import lax
from jax.experimental import pallas as pl
from jax.experimental.pallas import tpu as pltpu
from jax.experimental.pallas import tpu_sc as plsc

_INFO = plsc.get_sparse_core_info()
_NC = _INFO.num_cores
_NS = _INFO.num_subcores
_NW = _NC * _NS
_LN2 = 0.6931471805599453


@functools.cache
def _make_gather(n, v, chunk):
    per_w = n // _NW
    n_chunks = per_w // chunk
    assert n_chunks >= 2 and per_w % chunk == 0
    n_full = v // chunk
    tail_off = n_full * chunk
    tail = v - tail_off
    assert tail % 16 == 0
    plan = [[] for _ in range(_NS)]
    for c in range(n_full):
        plan[c % _NS].append((c * chunk, chunk))
    if tail:
        plan[_NS - 1].append((tail_off, tail))
    mesh = plsc.VectorSubcoreMesh(core_axis_name="c", subcore_axis_name="s")

    @functools.partial(
        pl.kernel,
        mesh=mesh,
        out_type=jax.ShapeDtypeStruct((n,), jnp.float32),
        scratch_types=[
            pltpu.VMEM_SHARED((v,), jnp.float32),
            pltpu.VMEM((chunk,), jnp.int32),
            pltpu.VMEM((chunk,), jnp.int32),
            pltpu.VMEM((chunk,), jnp.float32),
            pltpu.VMEM((chunk,), jnp.float32),
            pltpu.SemaphoreType.DMA,
            pltpu.SemaphoreType.DMA,
            pltpu.SemaphoreType.DMA,
            pltpu.SemaphoreType.DMA,
            pltpu.SemaphoreType.DMA,
            pltpu.SemaphoreType.DMA,
        ],
    )
    def gather_kernel(table_hbm, idx_hbm, out_hbm,
                      shared,
                      idx0, idx1, rows0, rows1,
                      si0, si1, sg0, sg1, so0, so1):
        tid = lax.axis_index("s")
        cid = lax.axis_index("c")
        wid = tid * _NC + cid
        base = wid * per_w
        idx_v = [idx0, idx1]
        rows_v = [rows0, rows1]
        si = [si0, si1]
        sg = [sg0, sg1]
        so = [so0, so1]
        fetches = [None, None]
        gathers = [None, None]
        stores = [None, None]

        for i in range(2):
            fetches[i] = pltpu.async_copy(
                idx_hbm.at[pl.ds(base + i * chunk, chunk)], idx_v[i], si[i])

        def stage(off, size):
            pltpu.sync_copy(table_hbm.at[0, pl.ds(off, size)],
                            rows0.at[pl.ds(0, size)])

            def body(k, carry):
                s = pl.ds(k * 16, 16)
                g = lax.iota(jnp.int32, 16) + (off + k * 16)
                rows0[s] = jnp.where(g == 0, 1.0, jnp.exp(rows0[s] * _LN2))
                return carry

            lax.fori_loop(0, size // 16, body, 0)
            pltpu.sync_copy(rows0.at[pl.ds(0, size)],
                            shared.at[pl.ds(off, size)])

        for t in range(_NS):
            @pl.when(tid == t)
            def _(t=t):
                for off, size in plan[t]:
                    stage(off, size)

        plsc.subcore_barrier()

        fetches[0].wait()
        gathers[0] = pltpu.async_copy(shared.at[idx_v[0]], rows_v[0], sg[0])

        for i in range(n_chunks):
            b, nb = i % 2, (i + 1) % 2
            if i + 1 < n_chunks:
                if stores[nb] is not None:
                    stores[nb].wait()
                fetches[nb].wait()
                gathers[nb] = pltpu.async_copy(
                    shared.at[idx_v[nb]], rows_v[nb], sg[nb])
            gathers[b].wait()
            stores[b] = pltpu.async_copy(
                rows_v[b], out_hbm.at[pl.ds(base + i * chunk, chunk)], so[b])
            if i + 2 < n_chunks:
                fetches[b] = pltpu.async_copy(
                    idx_hbm.at[pl.ds(base + (i + 2) * chunk, chunk)],
                    idx_v[b], si[b])
        stores[0].wait()
        stores[1].wait()

    return gather_kernel


def kernel(x, table):
    b, l = x.shape
    n = b * l
    v = table.shape[0]
    out = _make_gather(n, v, 12800)(table.T, x.T.reshape(-1))
    return out.reshape(l, b).transpose(1, 0)[..., None]
